# baseline (speedup 1.0000x reference)
import functools

import jax
import jax.numpy as jnp
from jax import lax
from jax.experimental import pallas as pl
from jax.experimental.pallas import tpu as pltpu
from jax.experimental.pallas import tpu_sc as plsc

_B = 16384
_EMB = 16
_NC = 2
_NS = 16
_NW = _NC * _NS
_NRING = 3

_mesh = plsc.VectorSubcoreMesh(core_axis_name="c", subcore_axis_name="s")


def _make_sc_gather(b):
    bpw = b // _NW
    ngrp = bpw // 16
    fbuf = min(128, bpw)
    fmask = fbuf // 16 - 1
    nring = _NRING

    @functools.partial(
        pl.kernel,
        mesh=_mesh,
        out_type=[jax.ShapeDtypeStruct((_EMB, b), jnp.float32)] * 4,
        scratch_types=[
            pltpu.VMEM((bpw,), jnp.int32),
            pltpu.VMEM((bpw,), jnp.int32),
            [pltpu.VMEM((_EMB, 128), jnp.float32)] * (16 * _NRING),
            pltpu.VMEM((_EMB, fbuf), jnp.float32),
            [pltpu.SemaphoreType.DMA] * _NRING,
        ],
        compiler_params=pltpu.CompilerParams(needs_layout_passes=False),
    )
    def _sc_gather(uidx_hbm, iidx_hbm, gu_hbm, gi_hbm, mu_hbm, mi_hbm,
                   gu_out, gi_out, mu_out, mi_out,
                   uidx_v, iidx_v, stages, rowbuf, sems):
        wid = lax.axis_index("s") * _NC + lax.axis_index("c")
        base = wid * bpw

        pltpu.sync_copy(uidx_hbm.at[wid], uidx_v)
        pltpu.sync_copy(iidx_hbm.at[wid], iidx_v)

        lanes = lax.iota(jnp.int32, 16)

        for table, idx_v, out in (
            (gu_hbm, uidx_v, gu_out),
            (gi_hbm, iidx_v, gi_out),
            (mu_hbm, uidx_v, mu_out),
            (mi_hbm, iidx_v, mi_out),
        ):
            def fire(g, ring, table=table, idx_v=idx_v):
                vec = idx_v[pl.ds(g * 16, 16)]
                for j in range(16):
                    u = vec[j]
                    bs = pl.multiple_of((u >> 7) * 128, 128)
                    pltpu.async_copy(
                        table.at[:, pl.ds(bs, 128)], stages[ring * 16 + j],
                        sems[ring])

            def extract(g, ring, table=table, idx_v=idx_v):
                for j in range(16):
                    pltpu.make_async_copy(
                        table.at[:, pl.ds(0, 128)], stages[ring * 16 + j],
                        sems[ring]).wait()
                vec = idx_v[pl.ds(g * 16, 16)]
                for j in range(16):
                    c = vec[j] & 127
                    val = plsc.load_gather(
                        stages[ring * 16 + j],
                        [lanes, jnp.zeros((16,), jnp.int32) + c])
                    pos = (g & fmask) * 16 + j
                    plsc.store_scatter(
                        rowbuf, [lanes, jnp.zeros((16,), jnp.int32) + pos],
                        val)

            for r in range(nring):
                fire(r, r)

            def body(h, carry, out=out, fire=fire, extract=extract):
                for k in range(nring):
                    g = nring * h + k

                    @pl.when(g < ngrp)
                    def _(g=g, k=k):
                        extract(g, k)

                        @pl.when(g + nring < ngrp)
                        def _():
                            fire(g + nring, k)

                        @pl.when((g & fmask) == fmask)
                        def _():
                            start = pl.multiple_of(
                                base + (g // (fbuf // 16)) * fbuf, fbuf)
                            pltpu.sync_copy(
                                rowbuf, out.at[:, pl.ds(start, fbuf)])

                return carry

            lax.fori_loop(0, (ngrp + nring - 1) // nring, body, 0)

    return _sc_gather


_sc_gather = _make_sc_gather(_B)


def _dense_body(gu, gi, xum, xim, gen, gWT, gb, W1T, b1, W2T, b2, WfT, bf,
                out):
    xg = jnp.dot(gWT[...], gen[...], preferred_element_type=jnp.float32) + gb[...]
    h = jnp.concatenate([xum[...], xim[...], xg], axis=0)
    h = jnp.maximum(
        jnp.dot(W1T[...], h, preferred_element_type=jnp.float32) + b1[...], 0.0)
    h = jnp.maximum(
        jnp.dot(W2T[...], h, preferred_element_type=jnp.float32) + b2[...], 0.0)
    wf = WfT[...]
    x_gmf = gu[...] * gi[...]
    acc = jnp.dot(wf[:, 0:_EMB], x_gmf, preferred_element_type=jnp.float32)
    acc = acc + jnp.dot(wf[:, _EMB:], h, preferred_element_type=jnp.float32)
    out[...] = acc + bf[...]


_BT = 2048


def _dense(gu, gi, xum, xim, gen, gWT, gb, W1T, b1, W2T, b2, WfT, bf):
    b = gu.shape[1]
    grid = (b // _BT,)
    col = lambda i: (0, i)
    full = lambda i: (0, 0)
    return pl.pallas_call(
        _dense_body,
        grid=grid,
        in_specs=[
            pl.BlockSpec((_EMB, _BT), col),
            pl.BlockSpec((_EMB, _BT), col),
            pl.BlockSpec((_EMB, _BT), col),
            pl.BlockSpec((_EMB, _BT), col),
            pl.BlockSpec((18, _BT), col),
            pl.BlockSpec((16, 18), full),
            pl.BlockSpec((16, 1), full),
            pl.BlockSpec((128, 48), full),
            pl.BlockSpec((128, 1), full),
            pl.BlockSpec((64, 128), full),
            pl.BlockSpec((64, 1), full),
            pl.BlockSpec((1, 80), full),
            pl.BlockSpec((1, 1), full),
        ],
        out_specs=pl.BlockSpec((1, _BT), col),
        out_shape=jax.ShapeDtypeStruct((1, b), jnp.float32),
        compiler_params=pltpu.CompilerParams(
            dimension_semantics=("parallel",)),
    )(gu, gi, xum, xim, gen, gWT, gb, W1T, b1, W2T, b2, WfT, bf)


def kernel(user_indices, item_indices, genres_vec, gmf_user_emb, gmf_item_emb,
           mlp_user_emb, mlp_item_emb, genres_W, genres_b, W1, b1, W2, b2,
           Wf, bf):
    ui = user_indices.astype(jnp.int32)
    ii = item_indices.astype(jnp.int32)
    tables = (gmf_user_emb.T, gmf_item_emb.T, mlp_user_emb.T,
              mlp_item_emb.T)
    dense_rest = (genres_W.T, genres_b.reshape(-1, 1), W1.T,
                  b1.reshape(-1, 1), W2.T, b2.reshape(-1, 1), Wf.T,
                  bf.reshape(-1, 1))
    rows = _sc_gather(
        ui.reshape(_NW, _B // _NW), ii.reshape(_NW, _B // _NW), *tables)
    out = _dense(*rows, genres_vec.T, *dense_rest)
    return out[0, :]


# speedup vs baseline: 1.0092x; 1.0092x over previous
import functools

import jax
import jax.numpy as jnp
from jax import lax
from jax.experimental import pallas as pl
from jax.experimental.pallas import tpu as pltpu
from jax.experimental.pallas import tpu_sc as plsc

_B = 16384
_EMB = 16
_NC = 2
_NS = 16
_NW = _NC * _NS
_NRING = 3

_mesh = plsc.VectorSubcoreMesh(core_axis_name="c", subcore_axis_name="s")


def _make_sc_gather(b):
    bpw = b // _NW
    ngrp = bpw // 16
    fbuf = min(128, bpw)
    fmask = fbuf // 16 - 1
    nring = _NRING

    @functools.partial(
        pl.kernel,
        mesh=_mesh,
        out_type=[jax.ShapeDtypeStruct((_EMB, b), jnp.float32)] * 4,
        scratch_types=[
            pltpu.VMEM((bpw,), jnp.int32),
            pltpu.VMEM((bpw,), jnp.int32),
            [pltpu.VMEM((_EMB, 128), jnp.float32)] * (16 * _NRING),
            pltpu.VMEM((_EMB, fbuf), jnp.float32),
            [pltpu.SemaphoreType.DMA] * _NRING,
        ],
        compiler_params=pltpu.CompilerParams(needs_layout_passes=False),
    )
    def _sc_gather(uidx_hbm, iidx_hbm, gu_hbm, gi_hbm, mu_hbm, mi_hbm,
                   gu_out, gi_out, mu_out, mi_out,
                   uidx_v, iidx_v, stages, rowbuf, sems):
        wid = lax.axis_index("s") * _NC + lax.axis_index("c")
        base = wid * bpw

        pltpu.sync_copy(uidx_hbm.at[wid], uidx_v)
        pltpu.sync_copy(iidx_hbm.at[wid], iidx_v)

        lanes = lax.iota(jnp.int32, 16)

        for table, idx_v, out in (
            (gu_hbm, uidx_v, gu_out),
            (gi_hbm, iidx_v, gi_out),
            (mu_hbm, uidx_v, mu_out),
            (mi_hbm, iidx_v, mi_out),
        ):
            def fire(g, ring, table=table, idx_v=idx_v):
                vec = idx_v[pl.ds(g * 16, 16)]
                for j in range(16):
                    u = vec[j]
                    bs = pl.multiple_of((u >> 7) * 128, 128)
                    pltpu.async_copy(
                        table.at[:, pl.ds(bs, 128)], stages[ring * 16 + j],
                        sems[ring])

            def extract(g, ring, table=table, idx_v=idx_v):
                for j in range(16):
                    pltpu.make_async_copy(
                        table.at[:, pl.ds(0, 128)], stages[ring * 16 + j],
                        sems[ring]).wait()
                vec = idx_v[pl.ds(g * 16, 16)]
                for j in range(16):
                    c = vec[j] & 127
                    val = plsc.load_gather(
                        stages[ring * 16 + j],
                        [lanes, jnp.zeros((16,), jnp.int32) + c])
                    pos = (g & fmask) * 16 + j
                    plsc.store_scatter(
                        rowbuf, [lanes, jnp.zeros((16,), jnp.int32) + pos],
                        val)

            for r in range(nring):
                fire(r, r)

            def body(h, carry, out=out, fire=fire, extract=extract):
                for k in range(nring):
                    g = nring * h + k

                    @pl.when(g < ngrp)
                    def _(g=g, k=k):
                        extract(g, k)

                        @pl.when(g + nring < ngrp)
                        def _():
                            fire(g + nring, k)

                        @pl.when((g & fmask) == fmask)
                        def _():
                            start = pl.multiple_of(
                                base + (g // (fbuf // 16)) * fbuf, fbuf)
                            pltpu.sync_copy(
                                rowbuf, out.at[:, pl.ds(start, fbuf)])

                return carry

            lax.fori_loop(0, (ngrp + nring - 1) // nring, body, 0)

    return _sc_gather


_sc_gather = _make_sc_gather(_B)


def _dense_body(gu, gi, xum, xim, gen, gWT, gb, W1T, b1, W2T, b2, WfT, bf,
                out):
    xg = jnp.dot(gWT[...], gen[...], preferred_element_type=jnp.float32) + gb[...]
    h = jnp.concatenate([xum[...], xim[...], xg], axis=0)
    h = jnp.maximum(
        jnp.dot(W1T[...], h, preferred_element_type=jnp.float32) + b1[...], 0.0)
    h = jnp.maximum(
        jnp.dot(W2T[...], h, preferred_element_type=jnp.float32) + b2[...], 0.0)
    wf = WfT[...]
    x_gmf = gu[...] * gi[...]
    acc = jnp.dot(wf[:, 0:_EMB], x_gmf, preferred_element_type=jnp.float32)
    acc = acc + jnp.dot(wf[:, _EMB:], h, preferred_element_type=jnp.float32)
    out[...] = acc + bf[...]


_BT = 4096


def _dense(gu, gi, xum, xim, gen, gWT, gb, W1T, b1, W2T, b2, WfT, bf):
    b = gu.shape[1]
    grid = (b // _BT,)
    col = lambda i: (0, i)
    full = lambda i: (0, 0)
    return pl.pallas_call(
        _dense_body,
        grid=grid,
        in_specs=[
            pl.BlockSpec((_EMB, _BT), col),
            pl.BlockSpec((_EMB, _BT), col),
            pl.BlockSpec((_EMB, _BT), col),
            pl.BlockSpec((_EMB, _BT), col),
            pl.BlockSpec((18, _BT), col),
            pl.BlockSpec((16, 18), full),
            pl.BlockSpec((16, 1), full),
            pl.BlockSpec((128, 48), full),
            pl.BlockSpec((128, 1), full),
            pl.BlockSpec((64, 128), full),
            pl.BlockSpec((64, 1), full),
            pl.BlockSpec((1, 80), full),
            pl.BlockSpec((1, 1), full),
        ],
        out_specs=pl.BlockSpec((1, _BT), col),
        out_shape=jax.ShapeDtypeStruct((1, b), jnp.float32),
        compiler_params=pltpu.CompilerParams(
            dimension_semantics=("parallel",)),
    )(gu, gi, xum, xim, gen, gWT, gb, W1T, b1, W2T, b2, WfT, bf)


def kernel(user_indices, item_indices, genres_vec, gmf_user_emb, gmf_item_emb,
           mlp_user_emb, mlp_item_emb, genres_W, genres_b, W1, b1, W2, b2,
           Wf, bf):
    ui = user_indices.astype(jnp.int32)
    ii = item_indices.astype(jnp.int32)
    tables = (gmf_user_emb.T, gmf_item_emb.T, mlp_user_emb.T,
              mlp_item_emb.T)
    dense_rest = (genres_W.T, genres_b.reshape(-1, 1), W1.T,
                  b1.reshape(-1, 1), W2.T, b2.reshape(-1, 1), Wf.T,
                  bf.reshape(-1, 1))
    rows = _sc_gather(
        ui.reshape(_NW, _B // _NW), ii.reshape(_NW, _B // _NW), *tables)
    out = _dense(*rows, genres_vec.T, *dense_rest)
    return out[0, :]


# speedup vs baseline: 1.0163x; 1.0070x over previous
import functools

import jax
import jax.numpy as jnp
from jax import lax
from jax.experimental import pallas as pl
from jax.experimental.pallas import tpu as pltpu
from jax.experimental.pallas import tpu_sc as plsc

_B = 16384
_EMB = 16
_NC = 2
_NS = 16
_NW = _NC * _NS
_NRING = 3

_mesh = plsc.VectorSubcoreMesh(core_axis_name="c", subcore_axis_name="s")


def _make_sc_gather(b):
    bpw = b // _NW
    ngrp = bpw // 16
    fbuf = min(128, bpw)
    fmask = fbuf // 16 - 1
    nring = _NRING

    @functools.partial(
        pl.kernel,
        mesh=_mesh,
        out_type=[jax.ShapeDtypeStruct((_EMB, b), jnp.float32)] * 4,
        scratch_types=[
            pltpu.VMEM((bpw,), jnp.int32),
            pltpu.VMEM((bpw,), jnp.int32),
            [pltpu.VMEM((_EMB, 128), jnp.float32)] * (16 * _NRING),
            pltpu.VMEM((_EMB, fbuf), jnp.float32),
            [pltpu.SemaphoreType.DMA] * _NRING,
        ],
        compiler_params=pltpu.CompilerParams(needs_layout_passes=False),
    )
    def _sc_gather(uidx_hbm, iidx_hbm, gu_hbm, gi_hbm, mu_hbm, mi_hbm,
                   gu_out, gi_out, mu_out, mi_out,
                   uidx_v, iidx_v, stages, rowbuf, sems):
        wid = lax.axis_index("s") * _NC + lax.axis_index("c")
        base = wid * bpw

        pltpu.sync_copy(uidx_hbm.at[wid], uidx_v)
        pltpu.sync_copy(iidx_hbm.at[wid], iidx_v)

        lanes = lax.iota(jnp.int32, 16)

        for table, idx_v, out in (
            (gu_hbm, uidx_v, gu_out),
            (gi_hbm, iidx_v, gi_out),
            (mu_hbm, uidx_v, mu_out),
            (mi_hbm, iidx_v, mi_out),
        ):
            def fire(g, ring, table=table, idx_v=idx_v):
                vec = idx_v[pl.ds(g * 16, 16)]
                for j in range(16):
                    u = vec[j]
                    bs = pl.multiple_of((u >> 7) * 128, 128)
                    pltpu.async_copy(
                        table.at[:, pl.ds(bs, 128)], stages[ring * 16 + j],
                        sems[ring])

            def extract(g, ring, table=table, idx_v=idx_v):
                for j in range(16):
                    pltpu.make_async_copy(
                        table.at[:, pl.ds(0, 128)], stages[ring * 16 + j],
                        sems[ring]).wait()
                vec = idx_v[pl.ds(g * 16, 16)]
                for j in range(16):
                    c = vec[j] & 127
                    val = plsc.load_gather(
                        stages[ring * 16 + j],
                        [lanes, jnp.zeros((16,), jnp.int32) + c])
                    pos = (g & fmask) * 16 + j
                    plsc.store_scatter(
                        rowbuf, [lanes, jnp.zeros((16,), jnp.int32) + pos],
                        val)

            for r in range(nring):
                fire(r, r)

            def body(h, carry, out=out, fire=fire, extract=extract):
                for k in range(nring):
                    g = nring * h + k

                    @pl.when(g < ngrp)
                    def _(g=g, k=k):
                        extract(g, k)

                        @pl.when(g + nring < ngrp)
                        def _():
                            fire(g + nring, k)

                        @pl.when((g & fmask) == fmask)
                        def _():
                            start = pl.multiple_of(
                                base + (g // (fbuf // 16)) * fbuf, fbuf)
                            pltpu.sync_copy(
                                rowbuf, out.at[:, pl.ds(start, fbuf)])

                return carry

            lax.fori_loop(0, (ngrp + nring - 1) // nring, body, 0)

    return _sc_gather


_sc_gather = _make_sc_gather(_B)


def _dense_body(gu, gi, xum, xim, gen, gWT, gb, W1T, b1, W2T, b2, WfT, bf,
                out):
    xg = jnp.dot(gWT[...], gen[...], preferred_element_type=jnp.float32) + gb[...]
    h = jnp.concatenate([xum[...], xim[...], xg], axis=0)
    h = jnp.maximum(
        jnp.dot(W1T[...], h, preferred_element_type=jnp.float32) + b1[...], 0.0)
    h = jnp.maximum(
        jnp.dot(W2T[...], h, preferred_element_type=jnp.float32) + b2[...], 0.0)
    wf = WfT[...]
    x_gmf = gu[...] * gi[...]
    acc = jnp.dot(wf[:, 0:_EMB], x_gmf, preferred_element_type=jnp.float32)
    acc = acc + jnp.dot(wf[:, _EMB:], h, preferred_element_type=jnp.float32)
    out[...] = acc + bf[...]


_BT = 8192


def _dense(gu, gi, xum, xim, gen, gWT, gb, W1T, b1, W2T, b2, WfT, bf):
    b = gu.shape[1]
    grid = (b // _BT,)
    col = lambda i: (0, i)
    full = lambda i: (0, 0)
    return pl.pallas_call(
        _dense_body,
        grid=grid,
        in_specs=[
            pl.BlockSpec((_EMB, _BT), col),
            pl.BlockSpec((_EMB, _BT), col),
            pl.BlockSpec((_EMB, _BT), col),
            pl.BlockSpec((_EMB, _BT), col),
            pl.BlockSpec((18, _BT), col),
            pl.BlockSpec((16, 18), full),
            pl.BlockSpec((16, 1), full),
            pl.BlockSpec((128, 48), full),
            pl.BlockSpec((128, 1), full),
            pl.BlockSpec((64, 128), full),
            pl.BlockSpec((64, 1), full),
            pl.BlockSpec((1, 80), full),
            pl.BlockSpec((1, 1), full),
        ],
        out_specs=pl.BlockSpec((1, _BT), col),
        out_shape=jax.ShapeDtypeStruct((1, b), jnp.float32),
        compiler_params=pltpu.CompilerParams(
            dimension_semantics=("parallel",)),
    )(gu, gi, xum, xim, gen, gWT, gb, W1T, b1, W2T, b2, WfT, bf)


def kernel(user_indices, item_indices, genres_vec, gmf_user_emb, gmf_item_emb,
           mlp_user_emb, mlp_item_emb, genres_W, genres_b, W1, b1, W2, b2,
           Wf, bf):
    ui = user_indices.astype(jnp.int32)
    ii = item_indices.astype(jnp.int32)
    tables = (gmf_user_emb.T, gmf_item_emb.T, mlp_user_emb.T,
              mlp_item_emb.T)
    dense_rest = (genres_W.T, genres_b.reshape(-1, 1), W1.T,
                  b1.reshape(-1, 1), W2.T, b2.reshape(-1, 1), Wf.T,
                  bf.reshape(-1, 1))
    rows = _sc_gather(
        ui.reshape(_NW, _B // _NW), ii.reshape(_NW, _B // _NW), *tables)
    out = _dense(*rows, genres_vec.T, *dense_rest)
    return out[0, :]
